# baseline (speedup 1.0000x reference)
import functools

import jax
import jax.numpy as jnp
from jax import lax
from jax.experimental import pallas as pl
from jax.experimental.pallas import tpu as pltpu
from jax.experimental.pallas import tpu_sc as plsc

B = 1024
E = 32
NC, NS, L = 2, 16, 16
NW = NC * NS
BPW = B // NW


def _sc_gather_dot(x0, x1, semb, sbias, femb, fbias):
    mesh = plsc.VectorSubcoreMesh(core_axis_name="c", subcore_axis_name="s")

    @functools.partial(
        pl.kernel,
        mesh=mesh,
        compiler_params=pltpu.CompilerParams(
            needs_layout_passes=False, use_tc_tiling_on_sc=False),
        out_type=[
            jax.ShapeDtypeStruct((B,), jnp.float32),
            jax.ShapeDtypeStruct((B,), jnp.float32),
        ],
        scratch_types=[
            pltpu.VMEM((BPW,), jnp.int32),
            pltpu.VMEM((BPW,), jnp.int32),
            pltpu.VMEM((BPW, E), jnp.float32),
            pltpu.VMEM((BPW, E), jnp.float32),
            pltpu.VMEM((BPW,), jnp.float32),
            pltpu.VMEM((BPW,), jnp.float32),
            pltpu.VMEM((BPW,), jnp.float32),
            pltpu.VMEM((BPW,), jnp.float32),
            pltpu.SemaphoreType.DMA,
        ],
    )
    def body(x0_h, x1_h, semb_h, sbias_h, femb_h, fbias_h, d_h, b_h,
             idx0_v, idx1_v, se_v, fe_v, sb_v, fb_v, dout_v, bout_v, sem):
        wid = lax.axis_index("s") * NC + lax.axis_index("c")
        base = wid * BPW
        pltpu.sync_copy(x0_h.at[pl.ds(base, BPW)], idx0_v)
        pltpu.sync_copy(x1_h.at[pl.ds(base, BPW)], idx1_v)
        cp1 = pltpu.async_copy(semb_h.at[idx0_v], se_v, sem)
        cp2 = pltpu.async_copy(femb_h.at[idx1_v], fe_v, sem)
        cp3 = pltpu.async_copy(sbias_h.at[idx0_v], sb_v, sem)
        cp4 = pltpu.async_copy(fbias_h.at[idx1_v], fb_v, sem)
        cp1.wait()
        cp2.wait()
        cp3.wait()
        cp4.wait()
        inv = jnp.float32(1.0 / E)
        lane = lax.iota(jnp.int32, L)
        for g in range(BPW // L):
            acc = jnp.zeros((L,), jnp.float32)
            for j in range(L):
                row = g * L + j
                p = (se_v[row, pl.ds(0, L)] * fe_v[row, pl.ds(0, L)]
                     + se_v[row, pl.ds(L, L)] * fe_v[row, pl.ds(L, L)])
                acc = jnp.where(lane == j, jnp.sum(p), acc)
            dout_v[pl.ds(g * L, L)] = acc * inv
        for h in range(BPW // L):
            sl = pl.ds(h * L, L)
            bout_v[sl] = sb_v[sl] + fb_v[sl]
        pltpu.sync_copy(dout_v, d_h.at[pl.ds(base, BPW)])
        pltpu.sync_copy(bout_v, b_h.at[pl.ds(base, BPW)])

    return body(x0, x1, semb, sbias, femb, fbias)


def _tc_broadcast_sigmoid(d_row, b_col):
    def body(b_ref, d_ref, o_ref):
        s = b_ref[...] + d_ref[...]
        o_ref[...] = 1.0 / (1.0 + jnp.exp(-s))

    return pl.pallas_call(
        body,
        grid=(8,),
        in_specs=[
            pl.BlockSpec((B // 8, 1), lambda i: (i, 0)),
            pl.BlockSpec((1, B), lambda i: (0, 0)),
        ],
        out_specs=pl.BlockSpec((B // 8, B), lambda i: (i, 0)),
        out_shape=jax.ShapeDtypeStruct((B, B), jnp.float32),
    )(b_col, d_row)


def kernel(x, sample_embedding, sample_bias, feature_embedding, feature_bias):
    x0 = x[:, 0].astype(jnp.int32)
    x1 = x[:, 1].astype(jnp.int32)
    d_vec, b_vec = _sc_gather_dot(
        x0, x1,
        sample_embedding, sample_bias.reshape(-1),
        feature_embedding, feature_bias.reshape(-1),
    )
    return _tc_broadcast_sigmoid(d_vec.reshape(1, B), b_vec.reshape(B, 1))
